# baseline (device time: 9835 ns/iter reference)
import jax
import jax.numpy as jnp
from jax import lax
from jax.experimental import pallas as pl
from jax.experimental.pallas import tpu as pltpu

N_DEV = 4
BLOCK_M = 256


def kernel(x):
    m_per, n = x.shape
    n_blocks = m_per // BLOCK_M
    mid = n_blocks // 2

    def body(x_ref, out_ref, buf, partial_ref, recv_ref,
             copy_sems, send_sems, recv_sems):
        my_pos = lax.axis_index("i")
        barrier_sem = pltpu.get_barrier_semaphore()

        def copy_blk(b, slot):
            return pltpu.make_async_copy(
                x_ref.at[pl.ds(b * BLOCK_M, BLOCK_M), :],
                buf.at[slot],
                copy_sems.at[slot],
            )

        def make_rdma(half, d):
            slot = half * (N_DEV - 1) + (d - 1)
            return pltpu.make_async_remote_copy(
                src_ref=partial_ref.at[pl.ds(half, 1)],
                dst_ref=recv_ref.at[pl.ds(slot, 1)],
                send_sem=send_sems.at[slot],
                recv_sem=recv_sems.at[slot],
                device_id=((my_pos + d) % N_DEV,),
                device_id_type=pl.DeviceIdType.MESH,
            )

        for d in range(1, N_DEV):
            pl.semaphore_signal(
                barrier_sem, inc=1,
                device_id=((my_pos + d) % N_DEV,),
                device_id_type=pl.DeviceIdType.MESH,
            )

        copy_blk(0, 0).start()
        copy_blk(1, 1).start()
        for b in range(n_blocks):
            cur = b % 2
            copy_blk(b, cur).wait()
            blk = jnp.sum(buf[cur], axis=0, keepdims=True)
            half = 0 if b < mid else 1
            if b == 0 or b == mid:
                partial_ref[half:half + 1, :] = blk
            else:
                partial_ref[half:half + 1, :] += blk
            if b + 2 < n_blocks:
                copy_blk(b + 2, cur).start()
            if b == mid - 1:
                pl.semaphore_wait(barrier_sem, N_DEV - 1)
                for d in range(1, N_DEV):
                    make_rdma(0, d).start()

        for d in range(1, N_DEV):
            make_rdma(1, d).start()
        for h in range(2):
            for d in range(1, N_DEV):
                make_rdma(h, d).wait()

        out_ref[...] = (
            partial_ref[0:1, :]
            + partial_ref[1:2, :]
            + jnp.sum(recv_ref[...], axis=0, keepdims=True)
        )

    return pl.pallas_call(
        body,
        out_shape=jax.ShapeDtypeStruct((1, n), x.dtype),
        in_specs=[pl.BlockSpec(memory_space=pltpu.MemorySpace.HBM)],
        out_specs=pl.BlockSpec(memory_space=pltpu.VMEM),
        scratch_shapes=[
            pltpu.VMEM((2, BLOCK_M, n), x.dtype),
            pltpu.VMEM((2, n), x.dtype),
            pltpu.VMEM((2 * (N_DEV - 1), n), x.dtype),
            pltpu.SemaphoreType.DMA((2,)),
            pltpu.SemaphoreType.DMA((2 * (N_DEV - 1),)),
            pltpu.SemaphoreType.DMA((2 * (N_DEV - 1),)),
        ],
        compiler_params=pltpu.CompilerParams(collective_id=0),
    )(x)


# device time: 8787 ns/iter; 1.1193x vs baseline; 1.1193x over previous
import jax
import jax.numpy as jnp
from jax import lax
from jax.experimental import pallas as pl
from jax.experimental.pallas import tpu as pltpu

N_DEV = 4
BLOCK_M = 512


def kernel(x):
    m_per, n = x.shape
    n_blocks = m_per // BLOCK_M

    def body(x_ref, out_ref, buf, partial_ref, recv_ref,
             copy_sems, send_sems, recv_sems):
        my_pos = lax.axis_index("i")
        barrier_sem = pltpu.get_barrier_semaphore()

        def copy_blk(b):
            return pltpu.make_async_copy(
                x_ref.at[pl.ds(b * BLOCK_M, BLOCK_M), :],
                buf.at[b],
                copy_sems.at[b],
            )

        def make_rdma(d):
            return pltpu.make_async_remote_copy(
                src_ref=partial_ref,
                dst_ref=recv_ref.at[pl.ds(d - 1, 1)],
                send_sem=send_sems.at[d - 1],
                recv_sem=recv_sems.at[d - 1],
                device_id=((my_pos + d) % N_DEV,),
                device_id_type=pl.DeviceIdType.MESH,
            )

        for d in range(1, N_DEV):
            pl.semaphore_signal(
                barrier_sem, inc=1,
                device_id=((my_pos + d) % N_DEV,),
                device_id_type=pl.DeviceIdType.MESH,
            )

        for b in range(n_blocks):
            copy_blk(b).start()
        for b in range(n_blocks):
            copy_blk(b).wait()
            blk = jnp.sum(buf[b], axis=0, keepdims=True)
            if b == 0:
                partial_ref[...] = blk
            else:
                partial_ref[...] += blk

        pl.semaphore_wait(barrier_sem, N_DEV - 1)
        rdmas = [make_rdma(d) for d in range(1, N_DEV)]
        for rdma in rdmas:
            rdma.start()
        for rdma in rdmas:
            rdma.wait()

        partial_ref[...] += jnp.sum(recv_ref[...], axis=0, keepdims=True)
        out_cp = pltpu.make_async_copy(partial_ref, out_ref, copy_sems.at[0])
        out_cp.start()
        out_cp.wait()

    return pl.pallas_call(
        body,
        out_shape=jax.ShapeDtypeStruct((1, n), x.dtype),
        in_specs=[pl.BlockSpec(memory_space=pltpu.MemorySpace.HBM)],
        out_specs=pl.BlockSpec(memory_space=pltpu.MemorySpace.HBM),
        scratch_shapes=[
            pltpu.VMEM((m_per // BLOCK_M, BLOCK_M, n), x.dtype),
            pltpu.VMEM((1, n), x.dtype),
            pltpu.VMEM((N_DEV - 1, n), x.dtype),
            pltpu.SemaphoreType.DMA((m_per // BLOCK_M,)),
            pltpu.SemaphoreType.DMA((N_DEV - 1,)),
            pltpu.SemaphoreType.DMA((N_DEV - 1,)),
        ],
        compiler_params=pltpu.CompilerParams(collective_id=0),
    )(pltpu.with_memory_space_constraint(x, pltpu.MemorySpace.HBM))
